# baseline (speedup 1.0000x reference)
import functools

import jax
import jax.numpy as jnp
from jax import lax
from jax.experimental import pallas as pl
from jax.experimental.pallas import tpu as pltpu
from jax.experimental.pallas import tpu_sc as plsc

_NC = 2
_NS = 16
_NW = _NC * _NS

_TM = 512


def _router_body(x_ref, wr_ref, br_ref, w_ref, i_ref, cnt_ref, crun_ref):
    @pl.when(pl.program_id(0) == 0)
    def _init():
        crun_ref[...] = jnp.zeros_like(crun_ref)

    xb = x_ref[...]
    logits = lax.dot_general(
        xb, wr_ref[...], (((1,), (1,)), ((), ())),
        preferred_element_type=jnp.float32,
    ) + br_ref[...]
    e_iota = lax.broadcasted_iota(jnp.int32, logits.shape, 1)
    m1 = jnp.max(logits, axis=1, keepdims=True)
    i1 = jnp.argmax(logits, axis=1).astype(jnp.int32)
    oh1 = e_iota == i1[:, None]
    l2 = jnp.where(oh1, -jnp.inf, logits)
    m2 = jnp.max(l2, axis=1, keepdims=True)
    i2 = jnp.argmax(l2, axis=1).astype(jnp.int32)
    b = jnp.exp(m2 - m1)
    w1 = 1.0 / (1.0 + b)
    w2 = 1.0 - w1

    ohA = oh1.astype(jnp.float32)
    ohB = (e_iota == i2[:, None]).astype(jnp.float32)
    ohS = ohA + ohB
    n = ohA.shape[0]
    r_io = lax.broadcasted_iota(jnp.int32, (n, n), 0)
    c_io = lax.broadcasted_iota(jnp.int32, (n, n), 1)
    tri = (c_io < r_io).astype(jnp.float32)
    pref = lax.dot_general(
        tri, ohS, (((1,), (0,)), ((), ())),
        preferred_element_type=jnp.float32,
    ) + crun_ref[...]
    rank_a = jnp.sum(ohA * pref, axis=1)
    rank_b = jnp.sum(ohB * pref, axis=1)
    new_run = crun_ref[...] + jnp.sum(ohS, axis=0, keepdims=True)
    crun_ref[...] = new_run
    cnt_ref[...] = new_run

    w_ref[...] = jnp.where(e_iota == 0, w1, jnp.where(e_iota == 1, w2, 0.0))
    i_ref[...] = jnp.where(
        e_iota == 0, i1[:, None],
        jnp.where(e_iota == 1, i2[:, None],
                  jnp.where(e_iota == 2, rank_a.astype(jnp.int32)[:, None],
                            jnp.where(e_iota == 3,
                                      rank_b.astype(jnp.int32)[:, None], 0))))


def _router(x_flat, Wr, br):
    BS, D = x_flat.shape
    E = Wr.shape[0]
    TMA = 512
    grid = (BS // TMA,)
    return pl.pallas_call(
        _router_body,
        grid=grid,
        in_specs=[
            pl.BlockSpec((TMA, D), lambda i: (i, 0)),
            pl.BlockSpec((E, D), lambda i: (0, 0)),
            pl.BlockSpec((1, E), lambda i: (0, 0)),
        ],
        out_specs=[
            pl.BlockSpec((TMA, E), lambda i: (i, 0)),
            pl.BlockSpec((TMA, E), lambda i: (i, 0)),
            pl.BlockSpec((1, E), lambda i: (0, 0)),
        ],
        out_shape=[
            jax.ShapeDtypeStruct((BS, E), jnp.float32),
            jax.ShapeDtypeStruct((BS, E), jnp.int32),
            jax.ShapeDtypeStruct((1, E), jnp.float32),
        ],
        scratch_shapes=[pltpu.VMEM((1, E), jnp.float32)],
        compiler_params=pltpu.CompilerParams(
            dimension_semantics=("arbitrary",),
        ),
    )(x_flat, Wr, br.reshape(1, E))


def _make_sc_gather(V, D, N, CH, dtype):
    assert N % _NW == 0
    rows_pw = N // _NW
    assert rows_pw % CH == 0 and rows_pw % 8 == 0
    n_ch = rows_pw // CH
    assert n_ch % 2 == 0 and n_ch >= 4
    mesh = plsc.VectorSubcoreMesh(
        core_axis_name="c", subcore_axis_name="s",
        num_cores=_NC, num_subcores=_NS,
    )

    @functools.partial(
        pl.kernel,
        out_type=jax.ShapeDtypeStruct((N, D), dtype),
        mesh=mesh,
        scratch_types=[
            pltpu.VMEM((rows_pw,), jnp.int32),
            pltpu.VMEM((2, CH, D), dtype),
            pltpu.SemaphoreType.DMA,
            pltpu.SemaphoreType.DMA,
            pltpu.SemaphoreType.DMA,
            pltpu.SemaphoreType.DMA,
        ],
    )
    def gather(table_hbm, idx_hbm, out_hbm, idx_v, rows_v, g0, g1, w0, w1):
        wid = lax.axis_index("s") * _NC + lax.axis_index("c")
        base = wid * rows_pw
        g_sems = (g0, g1)
        w_sems = (w0, w1)
        pltpu.sync_copy(idx_hbm.at[pl.ds(base, rows_pw)], idx_v)

        def fire_gather(j, b):
            pltpu.async_copy(
                table_hbm.at[idx_v.at[pl.ds(j * CH, CH)]], rows_v.at[b],
                g_sems[b])

        fire_gather(0, 0)
        fire_gather(1, 1)

        @pl.loop(0, n_ch, step=2)
        def _pair(i):
            for b in range(2):
                j = i + b
                pltpu.make_async_copy(
                    table_hbm.at[idx_v.at[pl.ds(0, CH)]], rows_v.at[b],
                    g_sems[b]).wait()
                pltpu.async_copy(
                    rows_v.at[b], out_hbm.at[pl.ds(base + j * CH, CH)],
                    w_sems[b])
            for b in range(2):
                j = i + b

                @pl.when(j + 2 < n_ch)
                def _refill():
                    pltpu.make_async_copy(
                        rows_v.at[b], out_hbm.at[pl.ds(base, CH)],
                        w_sems[b]).wait()
                    fire_gather(j + 2, b)

        for b in range(2):
            pltpu.make_async_copy(
                rows_v.at[b], out_hbm.at[pl.ds(base, CH)], w_sems[b]).wait()

    return gather


def _make_sc_dispatch(BS, D, P, TCH):
    assert BS % _NW == 0
    rows_pw = BS // _NW
    assert rows_pw % TCH == 0 and rows_pw % 8 == 0
    n_ch = rows_pw // TCH
    assert n_ch % 2 == 0 and n_ch >= 4 and TCH <= 128
    mesh = plsc.VectorSubcoreMesh(
        core_axis_name="c", subcore_axis_name="s",
        num_cores=_NC, num_subcores=_NS,
    )

    @functools.partial(
        pl.kernel,
        out_type=jax.ShapeDtypeStruct((P, D), jnp.float32),
        mesh=mesh,
        scratch_types=[
            pltpu.VMEM((2, TCH, D), jnp.float32),
            pltpu.VMEM((n_ch, TCH), jnp.int32),
            pltpu.VMEM((n_ch, TCH), jnp.int32),
            pltpu.SemaphoreType.DMA,
            pltpu.SemaphoreType.DMA,
            pltpu.SemaphoreType.DMA,
            pltpu.SemaphoreType.DMA,
        ],
    )
    def dispatch(x_hbm, pa_hbm, pb_hbm, xs_hbm, buf, da, db, g0, g1, s0, s1):
        wid = lax.axis_index("s") * _NC + lax.axis_index("c")
        base = wid * rows_pw
        g_sems = (g0, g1)
        s_sems = (s0, s1)
        pltpu.sync_copy(pa_hbm.at[pl.ds(wid * n_ch, n_ch), :], da)
        pltpu.sync_copy(pb_hbm.at[pl.ds(wid * n_ch, n_ch), :], db)

        def fire_load(j, b):
            pltpu.async_copy(
                x_hbm.at[pl.ds(base + j * TCH, TCH)], buf.at[b], g_sems[b])

        fire_load(0, 0)
        fire_load(1, 1)

        @pl.loop(0, n_ch, step=2)
        def _pair(i):
            for b in range(2):
                j = i + b
                pltpu.make_async_copy(
                    x_hbm.at[pl.ds(0, TCH)], buf.at[b], g_sems[b]).wait()
                pltpu.async_copy(buf.at[b], xs_hbm.at[da.at[j]], s_sems[b])
                pltpu.async_copy(buf.at[b], xs_hbm.at[db.at[j]], s_sems[b])
            for b in range(2):
                j = i + b

                @pl.when(j + 2 < n_ch)
                def _refill():
                    for _ in range(2):
                        pltpu.make_async_copy(
                            buf.at[b], xs_hbm.at[da.at[0]], s_sems[b]).wait()
                    fire_load(j + 2, b)

        for b in range(2):
            for _ in range(2):
                pltpu.make_async_copy(
                    buf.at[b], xs_hbm.at[da.at[0]], s_sems[b]).wait()

    return dispatch


def _expert_body(gid_ref, xs_ref, we_ref, be_ref, out_ref):
    g = gid_ref[pl.program_id(0)]
    xb = xs_ref[...]
    wg = we_ref[g]
    z = lax.dot_general(
        xb, wg, (((1,), (1,)), ((), ())),
        preferred_element_type=jnp.float32,
    ) + be_ref[g][None, :]
    out_ref[...] = z * (1.0 / (1.0 + jnp.exp(-z)))


def _expert_mm(xs, We, be, gids):
    P, D = xs.shape
    E = We.shape[0]
    ntiles = P // _TM
    return pl.pallas_call(
        _expert_body,
        grid=(ntiles,),
        in_specs=[
            pl.BlockSpec(memory_space=pltpu.SMEM),
            pl.BlockSpec((_TM, D), lambda i: (i, 0)),
            pl.BlockSpec((E, D, D), lambda i: (0, 0, 0)),
            pl.BlockSpec((E, D), lambda i: (0, 0)),
        ],
        out_specs=pl.BlockSpec((_TM, D), lambda i: (i, 0)),
        out_shape=jax.ShapeDtypeStruct((P, D), jnp.float32),
        compiler_params=pltpu.CompilerParams(
            dimension_semantics=("arbitrary",),
        ),
    )(gids, xs, We, be)


def _out_body(ga_ref, gb_ref, wa_ref, wb_ref, x_ref, wo_ref, bo_ref, g_ref,
              o_ref):
    wa = wa_ref[0, 0][:, None]
    wb = wb_ref[0, 0][:, None]
    c = ga_ref[...] * wa + gb_ref[...] * wb
    z = lax.dot_general(
        c, wo_ref[...], (((1,), (1,)), ((), ())),
        preferred_element_type=jnp.float32,
    ) + bo_ref[...]
    y = x_ref[...] + z
    ms = jnp.mean(y * y, axis=1, keepdims=True)
    o_ref[...] = g_ref[...] * (y * lax.rsqrt(ms + 1e-6))


def _out_proj(gab, wa, wb, x_flat, Wo, bo, g):
    BS, D = x_flat.shape
    TMD = 512
    nb = BS // TMD
    return pl.pallas_call(
        _out_body,
        grid=(nb,),
        in_specs=[
            pl.BlockSpec((TMD, D), lambda i: (i, 0)),
            pl.BlockSpec((TMD, D), lambda i, nb=nb: (i + nb, 0)),
            pl.BlockSpec((1, 1, TMD), lambda i: (i, 0, 0)),
            pl.BlockSpec((1, 1, TMD), lambda i: (i, 0, 0)),
            pl.BlockSpec((TMD, D), lambda i: (i, 0)),
            pl.BlockSpec((D, D), lambda i: (0, 0)),
            pl.BlockSpec((1, D), lambda i: (0, 0)),
            pl.BlockSpec((1, D), lambda i: (0, 0)),
        ],
        out_specs=pl.BlockSpec((TMD, D), lambda i: (i, 0)),
        out_shape=jax.ShapeDtypeStruct((BS, D), jnp.float32),
    )(gab, gab, wa.reshape(nb, 1, TMD), wb.reshape(nb, 1, TMD),
      x_flat, Wo, bo.reshape(1, D), g.reshape(1, D))


def kernel(x, Wr, br, We, be, Wo, bo, g):
    B, S, D = x.shape
    E = Wr.shape[0]
    K = 2
    BS = B * S
    A = BS * K
    P = A + E * _TM

    x_flat = x.reshape(BS, D)
    wts8, idx8, cnt = _router(x_flat, Wr, br)
    flat_w = wts8[:, :K]

    counts = cnt[0].astype(jnp.int32)
    c_pad = ((counts + _TM - 1) // _TM) * _TM
    tri8 = jnp.tril(jnp.ones((E, E), jnp.float32), -1)
    starts = (tri8 @ c_pad.astype(jnp.float32)).astype(jnp.int32)
    pos_a = starts[idx8[:, 0]] + idx8[:, 2]
    pos_b = starts[idx8[:, 1]] + idx8[:, 3]
    ntiles = P // _TM
    offs = jnp.arange(ntiles, dtype=jnp.int32) * _TM
    gids = jnp.sum((offs[:, None] >= starts[None, :]).astype(jnp.int32),
                   axis=1) - 1

    xs = _make_sc_dispatch(BS, D, P, 32)(
        x_flat, pos_a.reshape(BS // 32, 32), pos_b.reshape(BS // 32, 32))

    h = _expert_mm(xs, We, be, gids)

    pos_ab = jnp.concatenate([pos_a, pos_b])
    gab = _make_sc_gather(P, D, A, 32, jnp.float32)(h, pos_ab)

    out = _out_proj(gab, flat_w[:, 0], flat_w[:, 1], x_flat, Wo, bo, g)
    return out.reshape(B, S, D)


# speedup vs baseline: 1.0163x; 1.0163x over previous
import functools

import jax
import jax.numpy as jnp
from jax import lax
from jax.experimental import pallas as pl
from jax.experimental.pallas import tpu as pltpu
from jax.experimental.pallas import tpu_sc as plsc

_NC = 2
_NS = 16
_NW = _NC * _NS

_TM = 512


def _router_body(x_ref, wr_ref, br_ref, w_ref, i_ref, cnt_ref, crun_ref):
    @pl.when(pl.program_id(0) == 0)
    def _init():
        crun_ref[...] = jnp.zeros_like(crun_ref)

    xb = x_ref[...]
    logits = lax.dot_general(
        xb, wr_ref[...], (((1,), (1,)), ((), ())),
        preferred_element_type=jnp.float32,
    ) + br_ref[...]
    e_iota = lax.broadcasted_iota(jnp.int32, logits.shape, 1)
    m1 = jnp.max(logits, axis=1, keepdims=True)
    i1 = jnp.argmax(logits, axis=1).astype(jnp.int32)
    oh1 = e_iota == i1[:, None]
    l2 = jnp.where(oh1, -jnp.inf, logits)
    m2 = jnp.max(l2, axis=1, keepdims=True)
    i2 = jnp.argmax(l2, axis=1).astype(jnp.int32)
    b = jnp.exp(m2 - m1)
    w1 = 1.0 / (1.0 + b)
    w2 = 1.0 - w1

    ohA = oh1.astype(jnp.float32)
    ohB = (e_iota == i2[:, None]).astype(jnp.float32)
    ohS = ohA + ohB
    n = ohA.shape[0]
    r_io = lax.broadcasted_iota(jnp.int32, (n, n), 0)
    c_io = lax.broadcasted_iota(jnp.int32, (n, n), 1)
    tri = (c_io < r_io).astype(jnp.float32)
    pref = lax.dot_general(
        tri, ohS, (((1,), (0,)), ((), ())),
        preferred_element_type=jnp.float32,
    ) + crun_ref[...]
    rank_a = jnp.sum(ohA * pref, axis=1)
    rank_b = jnp.sum(ohB * pref, axis=1)
    new_run = crun_ref[...] + jnp.sum(ohS, axis=0, keepdims=True)
    crun_ref[...] = new_run
    cnt_ref[...] = new_run

    w_ref[...] = jnp.where(e_iota == 0, w1, jnp.where(e_iota == 1, w2, 0.0))
    i_ref[...] = jnp.where(
        e_iota == 0, i1[:, None],
        jnp.where(e_iota == 1, i2[:, None],
                  jnp.where(e_iota == 2, rank_a.astype(jnp.int32)[:, None],
                            jnp.where(e_iota == 3,
                                      rank_b.astype(jnp.int32)[:, None], 0))))


def _router(x_flat, Wr, br):
    BS, D = x_flat.shape
    E = Wr.shape[0]
    TMA = 512
    grid = (BS // TMA,)
    return pl.pallas_call(
        _router_body,
        grid=grid,
        in_specs=[
            pl.BlockSpec((TMA, D), lambda i: (i, 0)),
            pl.BlockSpec((E, D), lambda i: (0, 0)),
            pl.BlockSpec((1, E), lambda i: (0, 0)),
        ],
        out_specs=[
            pl.BlockSpec((TMA, E), lambda i: (i, 0)),
            pl.BlockSpec((TMA, E), lambda i: (i, 0)),
            pl.BlockSpec((1, E), lambda i: (0, 0)),
        ],
        out_shape=[
            jax.ShapeDtypeStruct((BS, E), jnp.float32),
            jax.ShapeDtypeStruct((BS, E), jnp.int32),
            jax.ShapeDtypeStruct((1, E), jnp.float32),
        ],
        scratch_shapes=[pltpu.VMEM((1, E), jnp.float32)],
        compiler_params=pltpu.CompilerParams(
            dimension_semantics=("arbitrary",),
        ),
    )(x_flat, Wr, br.reshape(1, E))


def _make_sc_gather(V, D, N, CH, dtype):
    assert N % _NW == 0
    rows_pw = N // _NW
    assert rows_pw % CH == 0 and rows_pw % 8 == 0
    n_ch = rows_pw // CH
    assert n_ch % 2 == 0 and n_ch >= 4
    mesh = plsc.VectorSubcoreMesh(
        core_axis_name="c", subcore_axis_name="s",
        num_cores=_NC, num_subcores=_NS,
    )

    @functools.partial(
        pl.kernel,
        out_type=jax.ShapeDtypeStruct((N, D), dtype),
        mesh=mesh,
        scratch_types=[
            pltpu.VMEM((rows_pw,), jnp.int32),
            pltpu.VMEM((2, CH, D), dtype),
            pltpu.SemaphoreType.DMA,
            pltpu.SemaphoreType.DMA,
            pltpu.SemaphoreType.DMA,
            pltpu.SemaphoreType.DMA,
        ],
    )
    def gather(table_hbm, idx_hbm, out_hbm, idx_v, rows_v, g0, g1, w0, w1):
        wid = lax.axis_index("s") * _NC + lax.axis_index("c")
        base = wid * rows_pw
        g_sems = (g0, g1)
        w_sems = (w0, w1)
        pltpu.sync_copy(idx_hbm.at[pl.ds(base, rows_pw)], idx_v)

        def fire_gather(j, b):
            pltpu.async_copy(
                table_hbm.at[idx_v.at[pl.ds(j * CH, CH)]], rows_v.at[b],
                g_sems[b])

        fire_gather(0, 0)
        fire_gather(1, 1)

        @pl.loop(0, n_ch, step=2)
        def _pair(i):
            for b in range(2):
                j = i + b
                pltpu.make_async_copy(
                    table_hbm.at[idx_v.at[pl.ds(0, CH)]], rows_v.at[b],
                    g_sems[b]).wait()
                pltpu.async_copy(
                    rows_v.at[b], out_hbm.at[pl.ds(base + j * CH, CH)],
                    w_sems[b])

                @pl.when(j + 2 < n_ch)
                def _refill():
                    pltpu.make_async_copy(
                        rows_v.at[b], out_hbm.at[pl.ds(base, CH)],
                        w_sems[b]).wait()
                    fire_gather(j + 2, b)

        for b in range(2):
            pltpu.make_async_copy(
                rows_v.at[b], out_hbm.at[pl.ds(base, CH)], w_sems[b]).wait()

    return gather


def _make_sc_dispatch(BS, D, P, TCH):
    assert BS % _NW == 0
    rows_pw = BS // _NW
    assert rows_pw % TCH == 0 and rows_pw % 8 == 0
    n_ch = rows_pw // TCH
    assert n_ch % 2 == 0 and n_ch >= 4 and TCH <= 128
    mesh = plsc.VectorSubcoreMesh(
        core_axis_name="c", subcore_axis_name="s",
        num_cores=_NC, num_subcores=_NS,
    )

    @functools.partial(
        pl.kernel,
        out_type=jax.ShapeDtypeStruct((P, D), jnp.float32),
        mesh=mesh,
        scratch_types=[
            pltpu.VMEM((2, TCH, D), jnp.float32),
            pltpu.VMEM((n_ch, TCH), jnp.int32),
            pltpu.VMEM((n_ch, TCH), jnp.int32),
            pltpu.SemaphoreType.DMA,
            pltpu.SemaphoreType.DMA,
            pltpu.SemaphoreType.DMA,
            pltpu.SemaphoreType.DMA,
        ],
    )
    def dispatch(x_hbm, pa_hbm, pb_hbm, xs_hbm, buf, da, db, g0, g1, s0, s1):
        wid = lax.axis_index("s") * _NC + lax.axis_index("c")
        base = wid * rows_pw
        g_sems = (g0, g1)
        s_sems = (s0, s1)
        pltpu.sync_copy(pa_hbm.at[pl.ds(wid * n_ch, n_ch), :], da)
        pltpu.sync_copy(pb_hbm.at[pl.ds(wid * n_ch, n_ch), :], db)

        def fire_load(j, b):
            pltpu.async_copy(
                x_hbm.at[pl.ds(base + j * TCH, TCH)], buf.at[b], g_sems[b])

        fire_load(0, 0)
        fire_load(1, 1)

        @pl.loop(0, n_ch, step=2)
        def _pair(i):
            for b in range(2):
                j = i + b
                pltpu.make_async_copy(
                    x_hbm.at[pl.ds(0, TCH)], buf.at[b], g_sems[b]).wait()
                pltpu.async_copy(buf.at[b], xs_hbm.at[da.at[j]], s_sems[b])
                pltpu.async_copy(buf.at[b], xs_hbm.at[db.at[j]], s_sems[b])

                @pl.when(j + 2 < n_ch)
                def _refill():
                    for _ in range(2):
                        pltpu.make_async_copy(
                            buf.at[b], xs_hbm.at[da.at[0]], s_sems[b]).wait()
                    fire_load(j + 2, b)

        for b in range(2):
            for _ in range(2):
                pltpu.make_async_copy(
                    buf.at[b], xs_hbm.at[da.at[0]], s_sems[b]).wait()

    return dispatch


def _expert_body(gid_ref, xs_ref, we_ref, be_ref, out_ref):
    g = gid_ref[pl.program_id(0)]
    xb = xs_ref[...]
    wg = we_ref[g]
    z = lax.dot_general(
        xb, wg, (((1,), (1,)), ((), ())),
        preferred_element_type=jnp.float32,
    ) + be_ref[g][None, :]
    out_ref[...] = z * (1.0 / (1.0 + jnp.exp(-z)))


def _expert_mm(xs, We, be, gids):
    P, D = xs.shape
    E = We.shape[0]
    ntiles = P // _TM
    return pl.pallas_call(
        _expert_body,
        grid=(ntiles,),
        in_specs=[
            pl.BlockSpec(memory_space=pltpu.SMEM),
            pl.BlockSpec((_TM, D), lambda i: (i, 0)),
            pl.BlockSpec((E, D, D), lambda i: (0, 0, 0)),
            pl.BlockSpec((E, D), lambda i: (0, 0)),
        ],
        out_specs=pl.BlockSpec((_TM, D), lambda i: (i, 0)),
        out_shape=jax.ShapeDtypeStruct((P, D), jnp.float32),
        compiler_params=pltpu.CompilerParams(
            dimension_semantics=("arbitrary",),
        ),
    )(gids, xs, We, be)


def _out_body(ga_ref, gb_ref, wa_ref, wb_ref, x_ref, wo_ref, bo_ref, g_ref,
              o_ref):
    wa = wa_ref[0, 0][:, None]
    wb = wb_ref[0, 0][:, None]
    c = ga_ref[...] * wa + gb_ref[...] * wb
    z = lax.dot_general(
        c, wo_ref[...], (((1,), (1,)), ((), ())),
        preferred_element_type=jnp.float32,
    ) + bo_ref[...]
    y = x_ref[...] + z
    ms = jnp.mean(y * y, axis=1, keepdims=True)
    o_ref[...] = g_ref[...] * (y * lax.rsqrt(ms + 1e-6))


def _out_proj(gab, wa, wb, x_flat, Wo, bo, g):
    BS, D = x_flat.shape
    TMD = 512
    nb = BS // TMD
    return pl.pallas_call(
        _out_body,
        grid=(nb,),
        in_specs=[
            pl.BlockSpec((TMD, D), lambda i: (i, 0)),
            pl.BlockSpec((TMD, D), lambda i, nb=nb: (i + nb, 0)),
            pl.BlockSpec((1, 1, TMD), lambda i: (i, 0, 0)),
            pl.BlockSpec((1, 1, TMD), lambda i: (i, 0, 0)),
            pl.BlockSpec((TMD, D), lambda i: (i, 0)),
            pl.BlockSpec((D, D), lambda i: (0, 0)),
            pl.BlockSpec((1, D), lambda i: (0, 0)),
            pl.BlockSpec((1, D), lambda i: (0, 0)),
        ],
        out_specs=pl.BlockSpec((TMD, D), lambda i: (i, 0)),
        out_shape=jax.ShapeDtypeStruct((BS, D), jnp.float32),
    )(gab, gab, wa.reshape(nb, 1, TMD), wb.reshape(nb, 1, TMD),
      x_flat, Wo, bo.reshape(1, D), g.reshape(1, D))


def kernel(x, Wr, br, We, be, Wo, bo, g):
    B, S, D = x.shape
    E = Wr.shape[0]
    K = 2
    BS = B * S
    A = BS * K
    P = A + E * _TM

    x_flat = x.reshape(BS, D)
    wts8, idx8, cnt = _router(x_flat, Wr, br)
    flat_w = wts8[:, :K]

    counts = cnt[0].astype(jnp.int32)
    c_pad = ((counts + _TM - 1) // _TM) * _TM
    tri8 = jnp.tril(jnp.ones((E, E), jnp.float32), -1)
    starts = (tri8 @ c_pad.astype(jnp.float32)).astype(jnp.int32)
    pos_a = starts[idx8[:, 0]] + idx8[:, 2]
    pos_b = starts[idx8[:, 1]] + idx8[:, 3]
    ntiles = P // _TM
    offs = jnp.arange(ntiles, dtype=jnp.int32) * _TM
    gids = jnp.sum((offs[:, None] >= starts[None, :]).astype(jnp.int32),
                   axis=1) - 1

    xs = _make_sc_dispatch(BS, D, P, 32)(
        x_flat, pos_a.reshape(BS // 32, 32), pos_b.reshape(BS // 32, 32))

    h = _expert_mm(xs, We, be, gids)

    pos_ab = jnp.concatenate([pos_a, pos_b])
    gab = _make_sc_gather(P, D, A, 32, jnp.float32)(h, pos_ab)

    out = _out_proj(gab, flat_w[:, 0], flat_w[:, 1], x_flat, Wo, bo, g)
    return out.reshape(B, S, D)


# speedup vs baseline: 1.0205x; 1.0042x over previous
import functools

import jax
import jax.numpy as jnp
from jax import lax
from jax.experimental import pallas as pl
from jax.experimental.pallas import tpu as pltpu
from jax.experimental.pallas import tpu_sc as plsc

_NC = 2
_NS = 16
_NW = _NC * _NS

_TM = 512


def _router_body(x_ref, wr_ref, br_ref, w_ref, i_ref, cnt_ref, crun_ref):
    @pl.when(pl.program_id(0) == 0)
    def _init():
        crun_ref[...] = jnp.zeros_like(crun_ref)

    xb = x_ref[...]
    logits = lax.dot_general(
        xb, wr_ref[...], (((1,), (1,)), ((), ())),
        preferred_element_type=jnp.float32,
    ) + br_ref[...]
    e_iota = lax.broadcasted_iota(jnp.int32, logits.shape, 1)
    m1 = jnp.max(logits, axis=1, keepdims=True)
    i1 = jnp.argmax(logits, axis=1).astype(jnp.int32)
    oh1 = e_iota == i1[:, None]
    l2 = jnp.where(oh1, -jnp.inf, logits)
    m2 = jnp.max(l2, axis=1, keepdims=True)
    i2 = jnp.argmax(l2, axis=1).astype(jnp.int32)
    b = jnp.exp(m2 - m1)
    w1 = 1.0 / (1.0 + b)
    w2 = 1.0 - w1

    ohA = oh1.astype(jnp.float32)
    ohB = (e_iota == i2[:, None]).astype(jnp.float32)
    ohS = ohA + ohB
    n = ohA.shape[0]
    r_io = lax.broadcasted_iota(jnp.int32, (n, n), 0)
    c_io = lax.broadcasted_iota(jnp.int32, (n, n), 1)
    tri = (c_io < r_io).astype(jnp.float32)
    pref = lax.dot_general(
        tri, ohS, (((1,), (0,)), ((), ())),
        preferred_element_type=jnp.float32,
    ) + crun_ref[...]
    rank_a = jnp.sum(ohA * pref, axis=1)
    rank_b = jnp.sum(ohB * pref, axis=1)
    new_run = crun_ref[...] + jnp.sum(ohS, axis=0, keepdims=True)
    crun_ref[...] = new_run
    cnt_ref[...] = new_run

    w_ref[...] = jnp.where(e_iota == 0, w1, jnp.where(e_iota == 1, w2, 0.0))
    i_ref[...] = jnp.where(
        e_iota == 0, i1[:, None],
        jnp.where(e_iota == 1, i2[:, None],
                  jnp.where(e_iota == 2, rank_a.astype(jnp.int32)[:, None],
                            jnp.where(e_iota == 3,
                                      rank_b.astype(jnp.int32)[:, None], 0))))


def _router(x_flat, Wr, br):
    BS, D = x_flat.shape
    E = Wr.shape[0]
    TMA = 512
    grid = (BS // TMA,)
    return pl.pallas_call(
        _router_body,
        grid=grid,
        in_specs=[
            pl.BlockSpec((TMA, D), lambda i: (i, 0)),
            pl.BlockSpec((E, D), lambda i: (0, 0)),
            pl.BlockSpec((1, E), lambda i: (0, 0)),
        ],
        out_specs=[
            pl.BlockSpec((TMA, E), lambda i: (i, 0)),
            pl.BlockSpec((TMA, E), lambda i: (i, 0)),
            pl.BlockSpec((1, E), lambda i: (0, 0)),
        ],
        out_shape=[
            jax.ShapeDtypeStruct((BS, E), jnp.float32),
            jax.ShapeDtypeStruct((BS, E), jnp.int32),
            jax.ShapeDtypeStruct((1, E), jnp.float32),
        ],
        scratch_shapes=[pltpu.VMEM((1, E), jnp.float32)],
        compiler_params=pltpu.CompilerParams(
            dimension_semantics=("arbitrary",),
        ),
    )(x_flat, Wr, br.reshape(1, E))


def _make_sc_gather(V, D, N, CH, dtype):
    assert N % _NW == 0
    rows_pw = N // _NW
    assert rows_pw % CH == 0 and rows_pw % 8 == 0
    n_ch = rows_pw // CH
    assert n_ch % 2 == 0 and n_ch >= 4
    mesh = plsc.VectorSubcoreMesh(
        core_axis_name="c", subcore_axis_name="s",
        num_cores=_NC, num_subcores=_NS,
    )

    @functools.partial(
        pl.kernel,
        out_type=jax.ShapeDtypeStruct((N, D), dtype),
        mesh=mesh,
        scratch_types=[
            pltpu.VMEM((rows_pw,), jnp.int32),
            pltpu.VMEM((2, CH, D), dtype),
            pltpu.SemaphoreType.DMA,
            pltpu.SemaphoreType.DMA,
            pltpu.SemaphoreType.DMA,
            pltpu.SemaphoreType.DMA,
        ],
    )
    def gather(table_hbm, idx_hbm, out_hbm, idx_v, rows_v, g0, g1, w0, w1):
        wid = lax.axis_index("s") * _NC + lax.axis_index("c")
        base = wid * rows_pw
        g_sems = (g0, g1)
        w_sems = (w0, w1)
        pltpu.sync_copy(idx_hbm.at[pl.ds(base, rows_pw)], idx_v)

        def fire_gather(j, b):
            pltpu.async_copy(
                table_hbm.at[idx_v.at[pl.ds(j * CH, CH)]], rows_v.at[b],
                g_sems[b])

        fire_gather(0, 0)
        fire_gather(1, 1)

        @pl.loop(0, n_ch, step=2)
        def _pair(i):
            for b in range(2):
                j = i + b
                pltpu.make_async_copy(
                    table_hbm.at[idx_v.at[pl.ds(0, CH)]], rows_v.at[b],
                    g_sems[b]).wait()
                pltpu.async_copy(
                    rows_v.at[b], out_hbm.at[pl.ds(base + j * CH, CH)],
                    w_sems[b])

                @pl.when(j + 2 < n_ch)
                def _refill():
                    pltpu.make_async_copy(
                        rows_v.at[b], out_hbm.at[pl.ds(base, CH)],
                        w_sems[b]).wait()
                    fire_gather(j + 2, b)

        for b in range(2):
            pltpu.make_async_copy(
                rows_v.at[b], out_hbm.at[pl.ds(base, CH)], w_sems[b]).wait()

    return gather


def _make_sc_dispatch(BS, D, P, TCH):
    assert BS % _NW == 0
    rows_pw = BS // _NW
    assert rows_pw % TCH == 0 and rows_pw % 8 == 0
    n_ch = rows_pw // TCH
    assert n_ch % 2 == 0 and n_ch >= 4 and TCH <= 128
    mesh = plsc.VectorSubcoreMesh(
        core_axis_name="c", subcore_axis_name="s",
        num_cores=_NC, num_subcores=_NS,
    )

    @functools.partial(
        pl.kernel,
        out_type=jax.ShapeDtypeStruct((P, D), jnp.float32),
        mesh=mesh,
        scratch_types=[
            pltpu.VMEM((2, TCH, D), jnp.float32),
            pltpu.VMEM((n_ch, TCH), jnp.int32),
            pltpu.VMEM((n_ch, TCH), jnp.int32),
            pltpu.SemaphoreType.DMA,
            pltpu.SemaphoreType.DMA,
            pltpu.SemaphoreType.DMA,
            pltpu.SemaphoreType.DMA,
        ],
    )
    def dispatch(x_hbm, pa_hbm, pb_hbm, xs_hbm, buf, da, db, g0, g1, s0, s1):
        wid = lax.axis_index("s") * _NC + lax.axis_index("c")
        base = wid * rows_pw
        g_sems = (g0, g1)
        s_sems = (s0, s1)
        pltpu.sync_copy(pa_hbm.at[pl.ds(wid * n_ch, n_ch), :], da)
        pltpu.sync_copy(pb_hbm.at[pl.ds(wid * n_ch, n_ch), :], db)

        def fire_load(j, b):
            pltpu.async_copy(
                x_hbm.at[pl.ds(base + j * TCH, TCH)], buf.at[b], g_sems[b])

        fire_load(0, 0)
        fire_load(1, 1)

        @pl.loop(0, n_ch, step=2)
        def _pair(i):
            for b in range(2):
                j = i + b
                pltpu.make_async_copy(
                    x_hbm.at[pl.ds(0, TCH)], buf.at[b], g_sems[b]).wait()
                pltpu.async_copy(buf.at[b], xs_hbm.at[da.at[j]], s_sems[b])
                pltpu.async_copy(buf.at[b], xs_hbm.at[db.at[j]], s_sems[b])

                @pl.when(j + 2 < n_ch)
                def _refill():
                    for _ in range(2):
                        pltpu.make_async_copy(
                            buf.at[b], xs_hbm.at[da.at[0]], s_sems[b]).wait()
                    fire_load(j + 2, b)

        for b in range(2):
            for _ in range(2):
                pltpu.make_async_copy(
                    buf.at[b], xs_hbm.at[da.at[0]], s_sems[b]).wait()

    return dispatch


def _expert_body(gid_ref, xs_ref, we_ref, be_ref, out_ref):
    g = gid_ref[pl.program_id(0)]
    xb = xs_ref[...]
    wg = we_ref[g]
    z = lax.dot_general(
        xb, wg, (((1,), (1,)), ((), ())),
        preferred_element_type=jnp.float32,
    ) + be_ref[g][None, :]
    out_ref[...] = z * (1.0 / (1.0 + jnp.exp(-z)))


def _expert_mm(xs, We, be, gids):
    P, D = xs.shape
    E = We.shape[0]
    ntiles = P // _TM
    return pl.pallas_call(
        _expert_body,
        grid=(ntiles,),
        in_specs=[
            pl.BlockSpec(memory_space=pltpu.SMEM),
            pl.BlockSpec((_TM, D), lambda i: (i, 0)),
            pl.BlockSpec((E, D, D), lambda i: (0, 0, 0)),
            pl.BlockSpec((E, D), lambda i: (0, 0)),
        ],
        out_specs=pl.BlockSpec((_TM, D), lambda i: (i, 0)),
        out_shape=jax.ShapeDtypeStruct((P, D), jnp.float32),
        compiler_params=pltpu.CompilerParams(
            dimension_semantics=("arbitrary",),
        ),
    )(gids, xs, We, be)


def _out_body(ga_ref, gb_ref, wa_ref, wb_ref, x_ref, wo_ref, bo_ref, g_ref,
              o_ref):
    wa = wa_ref[0, 0][:, None]
    wb = wb_ref[0, 0][:, None]
    c = ga_ref[...] * wa + gb_ref[...] * wb
    z = lax.dot_general(
        c, wo_ref[...], (((1,), (1,)), ((), ())),
        preferred_element_type=jnp.float32,
    ) + bo_ref[...]
    y = x_ref[...] + z
    ms = jnp.mean(y * y, axis=1, keepdims=True)
    o_ref[...] = g_ref[...] * (y * lax.rsqrt(ms + 1e-6))


def _out_proj(gab, wa, wb, x_flat, Wo, bo, g):
    BS, D = x_flat.shape
    TMD = 1024
    nb = BS // TMD
    return pl.pallas_call(
        _out_body,
        grid=(nb,),
        in_specs=[
            pl.BlockSpec((TMD, D), lambda i: (i, 0)),
            pl.BlockSpec((TMD, D), lambda i, nb=nb: (i + nb, 0)),
            pl.BlockSpec((1, 1, TMD), lambda i: (i, 0, 0)),
            pl.BlockSpec((1, 1, TMD), lambda i: (i, 0, 0)),
            pl.BlockSpec((TMD, D), lambda i: (i, 0)),
            pl.BlockSpec((D, D), lambda i: (0, 0)),
            pl.BlockSpec((1, D), lambda i: (0, 0)),
            pl.BlockSpec((1, D), lambda i: (0, 0)),
        ],
        out_specs=pl.BlockSpec((TMD, D), lambda i: (i, 0)),
        out_shape=jax.ShapeDtypeStruct((BS, D), jnp.float32),
    )(gab, gab, wa.reshape(nb, 1, TMD), wb.reshape(nb, 1, TMD),
      x_flat, Wo, bo.reshape(1, D), g.reshape(1, D))


def kernel(x, Wr, br, We, be, Wo, bo, g):
    B, S, D = x.shape
    E = Wr.shape[0]
    K = 2
    BS = B * S
    A = BS * K
    P = A + E * _TM

    x_flat = x.reshape(BS, D)
    wts8, idx8, cnt = _router(x_flat, Wr, br)
    flat_w = wts8[:, :K]

    counts = cnt[0].astype(jnp.int32)
    c_pad = ((counts + _TM - 1) // _TM) * _TM
    tri8 = jnp.tril(jnp.ones((E, E), jnp.float32), -1)
    starts = (tri8 @ c_pad.astype(jnp.float32)).astype(jnp.int32)
    pos_a = starts[idx8[:, 0]] + idx8[:, 2]
    pos_b = starts[idx8[:, 1]] + idx8[:, 3]
    ntiles = P // _TM
    offs = jnp.arange(ntiles, dtype=jnp.int32) * _TM
    gids = jnp.sum((offs[:, None] >= starts[None, :]).astype(jnp.int32),
                   axis=1) - 1

    xs = _make_sc_dispatch(BS, D, P, 32)(
        x_flat, pos_a.reshape(BS // 32, 32), pos_b.reshape(BS // 32, 32))

    h = _expert_mm(xs, We, be, gids)

    pos_ab = jnp.concatenate([pos_a, pos_b])
    gab = _make_sc_gather(P, D, A, 32, jnp.float32)(h, pos_ab)

    out = _out_proj(gab, flat_w[:, 0], flat_w[:, 1], x_flat, Wo, bo, g)
    return out.reshape(B, S, D)


# speedup vs baseline: 1.0301x; 1.0093x over previous
import functools

import jax
import jax.numpy as jnp
from jax import lax
from jax.experimental import pallas as pl
from jax.experimental.pallas import tpu as pltpu
from jax.experimental.pallas import tpu_sc as plsc

_NC = 2
_NS = 16
_NW = _NC * _NS

_TM = 512


def _router_body(x_ref, wr_ref, br_ref, w_ref, i_ref, cnt_ref, crun_ref):
    @pl.when(pl.program_id(0) == 0)
    def _init():
        crun_ref[...] = jnp.zeros_like(crun_ref)

    xb = x_ref[...]
    logits = lax.dot_general(
        xb, wr_ref[...], (((1,), (1,)), ((), ())),
        preferred_element_type=jnp.float32,
    ) + br_ref[...]
    e_iota = lax.broadcasted_iota(jnp.int32, logits.shape, 1)
    m1 = jnp.max(logits, axis=1, keepdims=True)
    i1 = jnp.argmax(logits, axis=1).astype(jnp.int32)
    oh1 = e_iota == i1[:, None]
    l2 = jnp.where(oh1, -jnp.inf, logits)
    m2 = jnp.max(l2, axis=1, keepdims=True)
    i2 = jnp.argmax(l2, axis=1).astype(jnp.int32)
    b = jnp.exp(m2 - m1)
    w1 = 1.0 / (1.0 + b)
    w2 = 1.0 - w1

    ohA = oh1.astype(jnp.float32)
    ohB = (e_iota == i2[:, None]).astype(jnp.float32)
    ohS = ohA + ohB
    n = ohA.shape[0]
    r_io = lax.broadcasted_iota(jnp.int32, (n, n), 0)
    c_io = lax.broadcasted_iota(jnp.int32, (n, n), 1)
    tri = (c_io < r_io).astype(jnp.float32)
    pref = lax.dot_general(
        tri, ohS, (((1,), (0,)), ((), ())),
        preferred_element_type=jnp.float32,
    ) + crun_ref[...]
    rank_a = jnp.sum(ohA * pref, axis=1)
    rank_b = jnp.sum(ohB * pref, axis=1)
    new_run = crun_ref[...] + jnp.sum(ohS, axis=0, keepdims=True)
    crun_ref[...] = new_run
    cnt_ref[...] = new_run

    w_ref[...] = jnp.where(e_iota == 0, w1, jnp.where(e_iota == 1, w2, 0.0))
    i_ref[...] = jnp.where(
        e_iota == 0, i1[:, None],
        jnp.where(e_iota == 1, i2[:, None],
                  jnp.where(e_iota == 2, rank_a.astype(jnp.int32)[:, None],
                            jnp.where(e_iota == 3,
                                      rank_b.astype(jnp.int32)[:, None], 0))))


def _router(x_flat, Wr, br):
    BS, D = x_flat.shape
    E = Wr.shape[0]
    TMA = 1024
    grid = (BS // TMA,)
    return pl.pallas_call(
        _router_body,
        grid=grid,
        in_specs=[
            pl.BlockSpec((TMA, D), lambda i: (i, 0)),
            pl.BlockSpec((E, D), lambda i: (0, 0)),
            pl.BlockSpec((1, E), lambda i: (0, 0)),
        ],
        out_specs=[
            pl.BlockSpec((TMA, E), lambda i: (i, 0)),
            pl.BlockSpec((TMA, E), lambda i: (i, 0)),
            pl.BlockSpec((1, E), lambda i: (0, 0)),
        ],
        out_shape=[
            jax.ShapeDtypeStruct((BS, E), jnp.float32),
            jax.ShapeDtypeStruct((BS, E), jnp.int32),
            jax.ShapeDtypeStruct((1, E), jnp.float32),
        ],
        scratch_shapes=[pltpu.VMEM((1, E), jnp.float32)],
        compiler_params=pltpu.CompilerParams(
            dimension_semantics=("arbitrary",),
        ),
    )(x_flat, Wr, br.reshape(1, E))


def _make_sc_gather(V, D, N, CH, dtype):
    assert N % _NW == 0
    rows_pw = N // _NW
    assert rows_pw % CH == 0 and rows_pw % 8 == 0
    n_ch = rows_pw // CH
    assert n_ch % 2 == 0 and n_ch >= 4
    mesh = plsc.VectorSubcoreMesh(
        core_axis_name="c", subcore_axis_name="s",
        num_cores=_NC, num_subcores=_NS,
    )

    @functools.partial(
        pl.kernel,
        out_type=jax.ShapeDtypeStruct((N, D), dtype),
        mesh=mesh,
        scratch_types=[
            pltpu.VMEM((rows_pw,), jnp.int32),
            pltpu.VMEM((2, CH, D), dtype),
            pltpu.SemaphoreType.DMA,
            pltpu.SemaphoreType.DMA,
            pltpu.SemaphoreType.DMA,
            pltpu.SemaphoreType.DMA,
        ],
    )
    def gather(table_hbm, idx_hbm, out_hbm, idx_v, rows_v, g0, g1, w0, w1):
        wid = lax.axis_index("s") * _NC + lax.axis_index("c")
        base = wid * rows_pw
        g_sems = (g0, g1)
        w_sems = (w0, w1)
        pltpu.sync_copy(idx_hbm.at[pl.ds(base, rows_pw)], idx_v)

        def fire_gather(j, b):
            pltpu.async_copy(
                table_hbm.at[idx_v.at[pl.ds(j * CH, CH)]], rows_v.at[b],
                g_sems[b])

        fire_gather(0, 0)
        fire_gather(1, 1)

        @pl.loop(0, n_ch, step=2)
        def _pair(i):
            for b in range(2):
                j = i + b
                pltpu.make_async_copy(
                    table_hbm.at[idx_v.at[pl.ds(0, CH)]], rows_v.at[b],
                    g_sems[b]).wait()
                pltpu.async_copy(
                    rows_v.at[b], out_hbm.at[pl.ds(base + j * CH, CH)],
                    w_sems[b])

                @pl.when(j + 2 < n_ch)
                def _refill():
                    pltpu.make_async_copy(
                        rows_v.at[b], out_hbm.at[pl.ds(base, CH)],
                        w_sems[b]).wait()
                    fire_gather(j + 2, b)

        for b in range(2):
            pltpu.make_async_copy(
                rows_v.at[b], out_hbm.at[pl.ds(base, CH)], w_sems[b]).wait()

    return gather


def _make_sc_dispatch(BS, D, P, TCH):
    assert BS % _NW == 0
    rows_pw = BS // _NW
    assert rows_pw % TCH == 0 and rows_pw % 8 == 0
    n_ch = rows_pw // TCH
    assert n_ch % 2 == 0 and n_ch >= 4 and TCH <= 128
    mesh = plsc.VectorSubcoreMesh(
        core_axis_name="c", subcore_axis_name="s",
        num_cores=_NC, num_subcores=_NS,
    )

    @functools.partial(
        pl.kernel,
        out_type=jax.ShapeDtypeStruct((P, D), jnp.float32),
        mesh=mesh,
        scratch_types=[
            pltpu.VMEM((2, TCH, D), jnp.float32),
            pltpu.VMEM((n_ch, TCH), jnp.int32),
            pltpu.VMEM((n_ch, TCH), jnp.int32),
            pltpu.SemaphoreType.DMA,
            pltpu.SemaphoreType.DMA,
            pltpu.SemaphoreType.DMA,
            pltpu.SemaphoreType.DMA,
        ],
    )
    def dispatch(x_hbm, pa_hbm, pb_hbm, xs_hbm, buf, da, db, g0, g1, s0, s1):
        wid = lax.axis_index("s") * _NC + lax.axis_index("c")
        base = wid * rows_pw
        g_sems = (g0, g1)
        s_sems = (s0, s1)
        pltpu.sync_copy(pa_hbm.at[pl.ds(wid * n_ch, n_ch), :], da)
        pltpu.sync_copy(pb_hbm.at[pl.ds(wid * n_ch, n_ch), :], db)

        def fire_load(j, b):
            pltpu.async_copy(
                x_hbm.at[pl.ds(base + j * TCH, TCH)], buf.at[b], g_sems[b])

        fire_load(0, 0)
        fire_load(1, 1)

        @pl.loop(0, n_ch, step=2)
        def _pair(i):
            for b in range(2):
                j = i + b
                pltpu.make_async_copy(
                    x_hbm.at[pl.ds(0, TCH)], buf.at[b], g_sems[b]).wait()
                pltpu.async_copy(buf.at[b], xs_hbm.at[da.at[j]], s_sems[b])
                pltpu.async_copy(buf.at[b], xs_hbm.at[db.at[j]], s_sems[b])

                @pl.when(j + 2 < n_ch)
                def _refill():
                    for _ in range(2):
                        pltpu.make_async_copy(
                            buf.at[b], xs_hbm.at[da.at[0]], s_sems[b]).wait()
                    fire_load(j + 2, b)

        for b in range(2):
            for _ in range(2):
                pltpu.make_async_copy(
                    buf.at[b], xs_hbm.at[da.at[0]], s_sems[b]).wait()

    return dispatch


def _expert_body(gid_ref, xs_ref, we_ref, be_ref, out_ref):
    g = gid_ref[pl.program_id(0)]
    xb = xs_ref[...]
    wg = we_ref[g]
    z = lax.dot_general(
        xb, wg, (((1,), (1,)), ((), ())),
        preferred_element_type=jnp.float32,
    ) + be_ref[g][None, :]
    out_ref[...] = z * (1.0 / (1.0 + jnp.exp(-z)))


def _expert_mm(xs, We, be, gids):
    P, D = xs.shape
    E = We.shape[0]
    ntiles = P // _TM
    return pl.pallas_call(
        _expert_body,
        grid=(ntiles,),
        in_specs=[
            pl.BlockSpec(memory_space=pltpu.SMEM),
            pl.BlockSpec((_TM, D), lambda i: (i, 0)),
            pl.BlockSpec((E, D, D), lambda i: (0, 0, 0)),
            pl.BlockSpec((E, D), lambda i: (0, 0)),
        ],
        out_specs=pl.BlockSpec((_TM, D), lambda i: (i, 0)),
        out_shape=jax.ShapeDtypeStruct((P, D), jnp.float32),
        compiler_params=pltpu.CompilerParams(
            dimension_semantics=("arbitrary",),
        ),
    )(gids, xs, We, be)


def _out_body(ga_ref, gb_ref, wa_ref, wb_ref, x_ref, wo_ref, bo_ref, g_ref,
              o_ref):
    wa = wa_ref[0, 0][:, None]
    wb = wb_ref[0, 0][:, None]
    c = ga_ref[...] * wa + gb_ref[...] * wb
    z = lax.dot_general(
        c, wo_ref[...], (((1,), (1,)), ((), ())),
        preferred_element_type=jnp.float32,
    ) + bo_ref[...]
    y = x_ref[...] + z
    ms = jnp.mean(y * y, axis=1, keepdims=True)
    o_ref[...] = g_ref[...] * (y * lax.rsqrt(ms + 1e-6))


def _out_proj(gab, wa, wb, x_flat, Wo, bo, g):
    BS, D = x_flat.shape
    TMD = 1024
    nb = BS // TMD
    return pl.pallas_call(
        _out_body,
        grid=(nb,),
        in_specs=[
            pl.BlockSpec((TMD, D), lambda i: (i, 0)),
            pl.BlockSpec((TMD, D), lambda i, nb=nb: (i + nb, 0)),
            pl.BlockSpec((1, 1, TMD), lambda i: (i, 0, 0)),
            pl.BlockSpec((1, 1, TMD), lambda i: (i, 0, 0)),
            pl.BlockSpec((TMD, D), lambda i: (i, 0)),
            pl.BlockSpec((D, D), lambda i: (0, 0)),
            pl.BlockSpec((1, D), lambda i: (0, 0)),
            pl.BlockSpec((1, D), lambda i: (0, 0)),
        ],
        out_specs=pl.BlockSpec((TMD, D), lambda i: (i, 0)),
        out_shape=jax.ShapeDtypeStruct((BS, D), jnp.float32),
    )(gab, gab, wa.reshape(nb, 1, TMD), wb.reshape(nb, 1, TMD),
      x_flat, Wo, bo.reshape(1, D), g.reshape(1, D))


def kernel(x, Wr, br, We, be, Wo, bo, g):
    B, S, D = x.shape
    E = Wr.shape[0]
    K = 2
    BS = B * S
    A = BS * K
    P = A + E * _TM

    x_flat = x.reshape(BS, D)
    wts8, idx8, cnt = _router(x_flat, Wr, br)
    flat_w = wts8[:, :K]

    counts = cnt[0].astype(jnp.int32)
    c_pad = ((counts + _TM - 1) // _TM) * _TM
    tri8 = jnp.tril(jnp.ones((E, E), jnp.float32), -1)
    starts = (tri8 @ c_pad.astype(jnp.float32)).astype(jnp.int32)
    pos_a = starts[idx8[:, 0]] + idx8[:, 2]
    pos_b = starts[idx8[:, 1]] + idx8[:, 3]
    ntiles = P // _TM
    offs = jnp.arange(ntiles, dtype=jnp.int32) * _TM
    gids = jnp.sum((offs[:, None] >= starts[None, :]).astype(jnp.int32),
                   axis=1) - 1

    xs = _make_sc_dispatch(BS, D, P, 32)(
        x_flat, pos_a.reshape(BS // 32, 32), pos_b.reshape(BS // 32, 32))

    h = _expert_mm(xs, We, be, gids)

    pos_ab = jnp.concatenate([pos_a, pos_b])
    gab = _make_sc_gather(P, D, A, 32, jnp.float32)(h, pos_ab)

    out = _out_proj(gab, flat_w[:, 0], flat_w[:, 1], x_flat, Wo, bo, g)
    return out.reshape(B, S, D)
